# baseline (speedup 1.0000x reference)
import functools
import math

import jax
import jax.numpy as jnp
from jax import lax
from jax.experimental import pallas as pl
from jax.experimental.pallas import tpu as pltpu
from jax.experimental.pallas import tpu_sc as plsc

NC = 2
NS = 16
LN = 16
NNODE = 10000
EDG = 320000
FD = 128
POOL_RATIO = 0.8

_NEG = -3.0e38


def _pad1024(v):
    return ((v + 1023) // 1024) * 1024


def _layer_sizes():
    out = []
    n = NNODE
    for _ in range(5):
        k = int(math.ceil(POOL_RATIO * n))
        out.append((n, k))
        n = k
    return out


LAYER_SIZES = _layer_sizes()
NPS = [_pad1024(n + 1) for (n, _) in LAYER_SIZES]
NPNS = [_pad1024(k + 1) for (_, k) in LAYER_SIZES]

_GATHER_DNUMS = lax.GatherDimensionNumbers(
    offset_dims=(), collapsed_slice_dims=(0,), start_index_map=(0,))


def _take16(v, idx):
    return lax.gather(v, idx[:, None], _GATHER_DNUMS, (1,),
                      mode=lax.GatherScatterMode.PROMISE_IN_BOUNDS)


def _lane_splat(vec, j):
    return _take16(vec, jnp.full((LN,), j, jnp.int32))


def _hsum16v(v):
    io = lax.iota(jnp.int32, LN)
    for sh in (8, 4, 2, 1):
        v = v + _take16(v, (io + sh) & (LN - 1))
    return v


def _last16(v):
    return _lane_splat(v, LN - 1)


def _cumsum16(v):
    io = lax.iota(jnp.int32, LN)
    zero = jnp.zeros_like(v)
    for sh in (1, 2, 4, 8):
        shifted = jnp.where(io >= sh, _take16(v, jnp.maximum(io - sh, 0)),
                            zero)
        v = v + shifted
    return v


def _rsqrt_newton(d):
    bits = lax.bitcast_convert_type(d, jnp.int32)
    y = lax.bitcast_convert_type(
        jnp.int32(0x5F3759DF) - lax.shift_right_arithmetic(bits, 1),
        jnp.float32)
    half = 0.5 * d
    for _ in range(3):
        y = y * (1.5 - half * y * y)
    return y


def _mono_key(z):
    b = lax.bitcast_convert_type(z, jnp.uint32)
    sign = lax.shift_right_logical(b, jnp.uint32(31))
    xorv = jnp.uint32(0x80000000) + sign * jnp.uint32(0x7FFFFFFF)
    return b ^ xorv


def _make_s1(n, NP):
    cpt = NP // NS
    RC = 16
    EC = 160
    ECD = 400
    EPT = EDG // NS
    NCH = EPT // EC
    NCHD = EPT // ECD
    NQ = FD // LN
    mesh = plsc.VectorSubcoreMesh(core_axis_name="c", subcore_axis_name="s",
                                  num_cores=1, num_subcores=NS)

    @functools.partial(
        pl.kernel,
        out_type=(
            jax.ShapeDtypeStruct((NP, FD), jnp.float32),
            jax.ShapeDtypeStruct((NP,), jnp.float32),
            jax.ShapeDtypeStruct((NP, FD), jnp.float32),
        ),
        mesh=mesh,
        compiler_params=pltpu.CompilerParams(needs_layout_passes=False),
        scratch_types=[
            pltpu.VMEM_SHARED((NP, FD), jnp.float32),
            pltpu.VMEM_SHARED((NP,), jnp.float32),
            pltpu.VMEM((RC, FD), jnp.float32),
            pltpu.VMEM((RC, FD), jnp.float32),
            pltpu.VMEM((cpt,), jnp.float32),
            pltpu.VMEM((RC,), jnp.float32),
            pltpu.VMEM((RC,), jnp.float32),
            pltpu.VMEM((EC,), jnp.int32),
            pltpu.VMEM((EC,), jnp.int32),
            pltpu.VMEM((EC,), jnp.int32),
            pltpu.VMEM((EC,), jnp.int32),
            pltpu.VMEM((EC, FD), jnp.float32),
            pltpu.VMEM((EC, FD), jnp.float32),
            pltpu.VMEM((ECD,), jnp.int32),
            pltpu.VMEM((ECD,), jnp.float32),
            pltpu.VMEM((FD,), jnp.float32),
            pltpu.SemaphoreType.DMA,
            pltpu.SemaphoreType.DMA,
        ],
    )
    def s1(rows_hbm, cols_hbm, xl_hbm, w_hbm, y_hbm, z_hbm, xls_hbm,
           acc_sp, deg_sp, tile0_v, tile1_v, dis_v, zc0_v, zc1_v,
           ri0_v, ri1_v, ci0_v, ci1_v, g0_v, g1_v, rid_v, one_v, w_v,
           gsem, ssem):
        s = lax.axis_index("s")
        rbase = s * cpt
        ebase = s * EPT
        zeros16 = jnp.zeros((LN,), jnp.float32)
        ones16 = jnp.ones((LN,), jnp.float32)
        iota16 = lax.iota(jnp.int32, LN)

        def _fill0(i, _):
            zc0_v[pl.ds(i * LN, LN)] = zeros16
            return 0
        lax.fori_loop(0, RC // LN, _fill0, 0)

        def _zdeg(h, _):
            pltpu.sync_copy(zc0_v, deg_sp.at[pl.ds(rbase + h * RC, RC)])
            return 0
        lax.fori_loop(0, cpt // RC, _zdeg, 0)

        def _fill1(i, _):
            one_v[pl.ds(i * LN, LN)] = ones16
            return 0
        lax.fori_loop(0, ECD // LN, _fill1, 0)
        pltpu.sync_copy(w_hbm, w_v)
        plsc.subcore_barrier()

        def _deg_chunk(j, _):
            pltpu.sync_copy(rows_hbm.at[pl.ds(ebase + j * ECD, ECD)], rid_v)

            def _rw(i, _2):
                r = rid_v[pl.ds(i * LN, LN)]
                spread = n + 1 + ((i * LN + iota16) & 127)
                rid_v[pl.ds(i * LN, LN)] = jnp.where(r == n, spread, r)
                return 0
            lax.fori_loop(0, ECD // LN, _rw, 0)
            pltpu.sync_copy(one_v, deg_sp.at[rid_v], add=True)
            return 0
        lax.fori_loop(0, NCHD, _deg_chunk, 0)
        plsc.subcore_barrier()

        pltpu.sync_copy(deg_sp.at[pl.ds(rbase, cpt)], dis_v)

        def _dis(i, _):
            d = dis_v[pl.ds(i * LN, LN)]
            gid = rbase + i * LN + iota16
            d = d + jnp.where(gid < n, 1.0, 0.0)
            dis_v[pl.ds(i * LN, LN)] = _rsqrt_newton(d)
            return 0
        lax.fori_loop(0, cpt // LN, _dis, 0)

        def _stage(h, _):
            lb2 = h * RC
            rb2 = rbase + lb2
            pltpu.sync_copy(xl_hbm.at[pl.ds(rb2, RC)], tile0_v)

            def _scale(i2, _2):
                dvec = dis_v[pl.ds(lb2 + i2 * LN, LN)]

                def _srow(jj, _3):
                    i = i2 * LN + jj
                    sd = _lane_splat(dvec, jj)
                    for q in range(NQ):
                        v = tile0_v[i, pl.ds(q * LN, LN)]
                        tile0_v[i, pl.ds(q * LN, LN)] = v * sd
                    return 0
                lax.fori_loop(0, LN, _srow, 0)
                return 0
            lax.fori_loop(0, RC // LN, _scale, 0)
            pltpu.sync_copy(tile0_v, xls_hbm.at[pl.ds(rb2, RC)])
            pltpu.sync_copy(tile0_v, acc_sp.at[pl.ds(rb2, RC)])
            return 0
        lax.fori_loop(0, cpt // RC, _stage, 0)
        plsc.subcore_barrier()

        bufs = ((ri0_v, ci0_v, g0_v), (ri1_v, ci1_v, g1_v))

        def _load_rewrite(j, ri_b, ci_b):
            pltpu.sync_copy(rows_hbm.at[pl.ds(ebase + j * EC, EC)], ri_b)
            pltpu.sync_copy(cols_hbm.at[pl.ds(ebase + j * EC, EC)], ci_b)

            def _rw(i, _2):
                r = ri_b[pl.ds(i * LN, LN)]
                cc = ci_b[pl.ds(i * LN, LN)]
                spread = n + 1 + ((i * LN + iota16) & 127)
                m = r == n
                ri_b[pl.ds(i * LN, LN)] = jnp.where(m, spread, r)
                ci_b[pl.ds(i * LN, LN)] = jnp.where(m, spread, cc)
                return 0
            lax.fori_loop(0, EC // LN, _rw, 0)

        def _iter(j, cur, nxt):
            ri_b, ci_b, g_b = cur
            ri_n, ci_n, g_n = nxt
            pltpu.make_async_copy(xls_hbm.at[ri_b], g_b, gsem).wait()
            pltpu.async_copy(g_b, acc_sp.at[ci_b], ssem, add=True)

            @pl.when(j + 1 < NCH)
            def _pref():
                @pl.when(j >= 1)
                def _ws():
                    pltpu.make_async_copy(g_n, acc_sp.at[ci_n], ssem).wait()
                _load_rewrite(j + 1, ri_n, ci_n)
                pltpu.async_copy(xls_hbm.at[ri_n], g_n, gsem)

        _load_rewrite(0, ri0_v, ci0_v)
        pltpu.async_copy(xls_hbm.at[ri0_v], g0_v, gsem)

        def _agg_chunk(j, _):
            @pl.when(j % 2 == 0)
            def _even():
                _iter(j, bufs[0], bufs[1])

            @pl.when(j % 2 == 1)
            def _odd():
                _iter(j, bufs[1], bufs[0])
            return 0
        lax.fori_loop(0, NCH, _agg_chunk, 0)
        pltpu.make_async_copy(g0_v, acc_sp.at[ci0_v], ssem).wait()
        pltpu.make_async_copy(g1_v, acc_sp.at[ci1_v], ssem).wait()
        plsc.subcore_barrier()

        w_regs = [w_v[pl.ds(q * LN, LN)] for q in range(NQ)]

        def _final(h, _):
            lb2 = h * RC
            rb2 = rbase + lb2
            pltpu.sync_copy(acc_sp.at[pl.ds(rb2, RC)], tile0_v)

            def _fin(i2, _2):
                dvec = dis_v[pl.ds(lb2 + i2 * LN, LN)]

                def _frow(jj, zacc):
                    i = i2 * LN + jj
                    gid = rb2 + i
                    sd = _lane_splat(dvec, jj)
                    dot = zeros16
                    for q in range(NQ):
                        v = tile0_v[i, pl.ds(q * LN, LN)]
                        v = jnp.maximum(v * sd, 0.0)
                        tile0_v[i, pl.ds(q * LN, LN)] = v
                        dot = dot + v * w_regs[q]
                    zval = jnp.where(gid < n, _hsum16v(dot),
                                     jnp.full((LN,), _NEG, jnp.float32))
                    return jnp.where(iota16 == jj, zval, zacc)
                zacc = lax.fori_loop(0, LN, _frow,
                                     jnp.full((LN,), _NEG, jnp.float32))
                zc0_v[pl.ds(i2 * LN, LN)] = zacc
                return 0
            lax.fori_loop(0, RC // LN, _fin, 0)
            pltpu.sync_copy(tile0_v, y_hbm.at[pl.ds(rb2, RC)])
            pltpu.sync_copy(zc0_v, z_hbm.at[pl.ds(rb2, RC)])
            return 0
        lax.fori_loop(0, cpt // RC, _final, 0)

    return s1


def _make_s2(n, k, NP, NPn):
    cpk = NPn // (NC * NS)
    CC = 32
    NCC = cpk // CC
    CE = 2000
    EPT2 = EDG // (NC * NS)
    NCH2 = EPT2 // CE
    NQ = FD // LN
    mesh = plsc.VectorSubcoreMesh(core_axis_name="c", subcore_axis_name="s",
                                  num_cores=NC, num_subcores=NS)

    @functools.partial(
        pl.kernel,
        out_type=(
            jax.ShapeDtypeStruct((NPn, FD), jnp.float32),
            jax.ShapeDtypeStruct((EDG,), jnp.int32),
            jax.ShapeDtypeStruct((EDG,), jnp.int32),
            jax.ShapeDtypeStruct((NC, FD), jnp.float32),
            jax.ShapeDtypeStruct((NC, FD), jnp.float32),
        ),
        mesh=mesh,
        compiler_params=pltpu.CompilerParams(needs_layout_passes=False),
        scratch_types=[
            pltpu.VMEM_SHARED((NS, FD), jnp.float32),
            pltpu.VMEM_SHARED((NS, FD), jnp.float32),
            pltpu.VMEM((NP,), jnp.float32),
            pltpu.VMEM((NP,), jnp.uint32),
            pltpu.VMEM((4096,), jnp.int32),
            pltpu.VMEM((NPn,), jnp.int32),
            pltpu.VMEM((NP,), jnp.int32),
            pltpu.VMEM((cpk,), jnp.float32),
            pltpu.VMEM((CC, FD), jnp.float32),
            pltpu.VMEM((CE,), jnp.int32),
            pltpu.VMEM((CE,), jnp.int32),
            pltpu.VMEM((FD,), jnp.float32),
            pltpu.VMEM((NS, FD), jnp.float32),
            pltpu.VMEM((LN,), jnp.float32),
        ],
    )
    def s2(z_hbm, y_hbm, rows_hbm, cols_hbm, winv_hbm,
           xn_hbm, rn_hbm, cn_hbm, pmax_hbm, psum_hbm,
           mx_sp, sm_sp, z_v, key_v, subh_v, sel_v, map_v, sc_v,
           xb_v, er_v, ec_v, ro_v, mst_v, wv_v):
        c = lax.axis_index("c")
        s = lax.axis_index("s")
        w = s * NC + c
        iota16 = lax.iota(jnp.int32, LN)
        ones16i = jnp.ones((LN,), jnp.int32)
        zeros16i = jnp.zeros((LN,), jnp.int32)

        pltpu.sync_copy(z_hbm, z_v)
        pltpu.sync_copy(winv_hbm.at[0].at[pl.ds(0, LN)], wv_v)

        def _keys(i, _):
            key_v[pl.ds(i * LN, LN)] = _mono_key(z_v[pl.ds(i * LN, LN)])
            return 0
        lax.fori_loop(0, NP // LN, _keys, 0)

        er_v[pl.ds(0, LN)] = ones16i
        ones_m = er_v[pl.ds(0, LN)]

        prefix = jnp.zeros((LN,), jnp.uint32)
        below = jnp.zeros((LN,), jnp.int32)
        q_eq = jnp.zeros((LN,), jnp.int32)
        target = jnp.full((LN,), NP - k, jnp.int32)
        for p in range(4):
            shift = 24 - 8 * p

            def _zeroh(i, _):
                subh_v[pl.ds(i * LN, LN)] = zeros16i
                return 0
            lax.fori_loop(0, 4096 // LN, _zeroh, 0)

            pref = prefix

            def _hist(i, _, _p=p, _sh=shift, _pref=pref):
                kv = key_v[pl.ds(i * LN, LN)]
                field = lax.convert_element_type(
                    lax.shift_right_logical(kv, jnp.uint32(_sh))
                    & jnp.uint32(255), jnp.int32)
                addr = field * LN + iota16
                if _p == 0:
                    plsc.addupdate_scatter(subh_v, [addr], ones_m)
                else:
                    match = lax.shift_right_logical(
                        kv, jnp.uint32(_sh + 8)) == lax.shift_right_logical(
                            _pref, jnp.uint32(_sh + 8))
                    plsc.addupdate_scatter(subh_v, [addr], ones_m, mask=match)
                return 0
            lax.fori_loop(0, NP // LN, _hist, 0)

            def _walk(b, st):
                below_, vstar_, done_, q_ = st
                hv = subh_v[pl.ds(b * LN, LN)]
                sb = _hsum16v(hv)
                cross = jnp.logical_and(jnp.logical_not(done_),
                                        below_ + sb > target)
                vstar_ = jnp.where(cross, b, vstar_)
                q_ = jnp.where(cross, sb, q_)
                below_ = jnp.where(jnp.logical_or(done_, cross), below_,
                                   below_ + sb)
                done_ = jnp.logical_or(done_, cross)
                return below_, vstar_, done_, q_
            below, vstar, _, q_eq = lax.fori_loop(
                0, 256, _walk,
                (below, jnp.zeros((LN,), jnp.int32),
                 jnp.zeros((LN,), jnp.bool_), q_eq))
            prefix = prefix | lax.shift_left(
                lax.convert_element_type(vstar, jnp.uint32), jnp.uint32(shift))
        tkey = prefix
        n_gt = jnp.full((LN,), NP, jnp.int32) - below - q_eq
        need_eq = jnp.full((LN,), k, jnp.int32) - n_gt

        def _pref0(i, _):
            sel_v[pl.ds(i * LN, LN)] = zeros16i
            return 0
        lax.fori_loop(0, NPn // LN, _pref0, 0)

        def _compact(i, st):
            nsel, neq = st
            kv = key_v[pl.ds(i * LN, LN)]
            m_gt = kv > tkey
            m_eq = kv == tkey
            ceq = _cumsum16(jnp.where(m_eq, 1, 0))
            take = jnp.logical_or(
                m_gt, jnp.logical_and(m_eq, neq + ceq <= need_eq))
            ctk = _cumsum16(jnp.where(take, 1, 0))
            pos = nsel + ctk - 1
            plsc.store_scatter(sel_v, [pos], i * LN + iota16, mask=take)
            return nsel + _last16(ctk), neq + _last16(ceq)
        lax.fori_loop(0, NP // LN, _compact,
                      (jnp.zeros((LN,), jnp.int32),
                       jnp.zeros((LN,), jnp.int32)))

        winv = _lane_splat(wv_v[...], 0)
        rb = w * cpk

        def _score(i, _):
            pos = rb + i * LN + iota16
            posc = jnp.minimum(pos, NPn - 1)
            sidx = plsc.load_gather(sel_v, [posc])
            zv = plsc.load_gather(z_v, [sidx])
            u = zv * winv
            t = jnp.exp(-2.0 * jnp.abs(u))
            th = (1.0 - t) / (1.0 + t)
            sc = jnp.where(u < 0.0, -th, th)
            sc = jnp.where(pos < k, sc, 0.0)
            sc_v[pl.ds(i * LN, LN)] = sc
            return 0
        lax.fori_loop(0, cpk // LN, _score, 0)

        carry0 = tuple([jnp.full((LN,), _NEG, jnp.float32)] * NQ
                       + [jnp.zeros((LN,), jnp.float32)] * NQ)

        def _chunk(m, carry):
            base = rb + m * CC
            pltpu.sync_copy(y_hbm.at[sel_v.at[pl.ds(base, CC)]], xb_v)

            def _row(j, st_):
                st_ = list(st_)
                scvec = sc_v[pl.ds((m * CC + j) // LN * LN, LN)]
                sc = _lane_splat(scvec, j % LN)
                real = base + j < k
                for q in range(NQ):
                    v = xb_v[j, pl.ds(q * LN, LN)] * sc
                    xb_v[j, pl.ds(q * LN, LN)] = v
                    st_[q] = jnp.where(real, jnp.maximum(st_[q], v), st_[q])
                    st_[NQ + q] = jnp.where(real, st_[NQ + q] + v,
                                            st_[NQ + q])
                return tuple(st_)
            st = lax.fori_loop(0, CC, _row, carry)
            pltpu.sync_copy(xb_v, xn_hbm.at[pl.ds(base, CC)])
            return st
        carry = lax.fori_loop(0, NCC, _chunk, carry0)
        carry = list(carry)

        for q in range(NQ):
            ro_v[pl.ds(q * LN, LN)] = carry[q]
        pltpu.sync_copy(ro_v, mx_sp.at[s])
        for q in range(NQ):
            ro_v[pl.ds(q * LN, LN)] = carry[NQ + q]
        pltpu.sync_copy(ro_v, sm_sp.at[s])

        neg16 = jnp.full((LN,), -1, jnp.int32)

        def _minit(i, _):
            map_v[pl.ds(i * LN, LN)] = neg16
            return 0
        lax.fori_loop(0, NP // LN, _minit, 0)

        def _mset(i, _):
            sidx = sel_v[pl.ds(i * LN, LN)]
            newid = i * LN + iota16
            plsc.store_scatter(map_v, [sidx], newid, mask=newid < k)
            return 0
        lax.fori_loop(0, (k + LN - 1) // LN, _mset, 0)

        def _remap(j, _):
            eb = w * EPT2 + j * CE
            pltpu.sync_copy(rows_hbm.at[pl.ds(eb, CE)], er_v)
            pltpu.sync_copy(cols_hbm.at[pl.ds(eb, CE)], ec_v)

            def _rm(i, _2):
                r = er_v[pl.ds(i * LN, LN)]
                cc2 = ec_v[pl.ds(i * LN, LN)]
                mr = plsc.load_gather(map_v, [r])
                mc = plsc.load_gather(map_v, [cc2])
                valid = jnp.logical_and(mr >= 0, mc >= 0)
                er_v[pl.ds(i * LN, LN)] = jnp.where(valid, mr, k)
                ec_v[pl.ds(i * LN, LN)] = jnp.where(valid, mc, k)
                return 0
            lax.fori_loop(0, CE // LN, _rm, 0)
            pltpu.sync_copy(er_v, rn_hbm.at[pl.ds(eb, CE)])
            pltpu.sync_copy(ec_v, cn_hbm.at[pl.ds(eb, CE)])
            return 0
        lax.fori_loop(0, NCH2, _remap, 0)

        plsc.subcore_barrier()

        @pl.when(s == 0)
        def _merge():
            pltpu.sync_copy(mx_sp, mst_v)

            def _redmax(r, st_):
                st_ = list(st_)
                for q in range(NQ):
                    st_[q] = jnp.maximum(st_[q], mst_v[r, pl.ds(q * LN, LN)])
                return tuple(st_)
            accm = lax.fori_loop(
                0, NS, _redmax,
                tuple([jnp.full((LN,), _NEG, jnp.float32)] * NQ))
            for q in range(NQ):
                ro_v[pl.ds(q * LN, LN)] = accm[q]
            pltpu.sync_copy(ro_v, pmax_hbm.at[c])

            pltpu.sync_copy(sm_sp, mst_v)

            def _redsum(r, st_):
                st_ = list(st_)
                for q in range(NQ):
                    st_[q] = st_[q] + mst_v[r, pl.ds(q * LN, LN)]
                return tuple(st_)
            accs = lax.fori_loop(
                0, NS, _redsum,
                tuple([jnp.zeros((LN,), jnp.float32)] * NQ))
            for q in range(NQ):
                ro_v[pl.ds(q * LN, LN)] = accs[q]
            pltpu.sync_copy(ro_v, psum_hbm.at[c])

    return s2


def _make_tmm(n, NP):
    def body(x_ref, wt_ref, b_ref, wp_ref, out_ref, winv_ref):
        i = pl.program_id(0)
        acc = jnp.dot(x_ref[...], wt_ref[...],
                      preferred_element_type=jnp.float32) + b_ref[...]
        rid = i * 128 + lax.broadcasted_iota(jnp.int32, (128, FD), 0)
        out_ref[...] = jnp.where(rid < n, acc, 0.0)

        @pl.when(i == 0)
        def _():
            wp = wp_ref[...]
            winv_ref[...] = jnp.full((1, FD),
                                     lax.rsqrt(jnp.sum(wp * wp)), jnp.float32)

    return pl.pallas_call(
        body,
        grid=(NP // 128,),
        in_specs=[
            pl.BlockSpec((128, FD), lambda i: (i, 0)),
            pl.BlockSpec((FD, FD), lambda i: (0, 0)),
            pl.BlockSpec((1, FD), lambda i: (0, 0)),
            pl.BlockSpec((1, FD), lambda i: (0, 0)),
        ],
        out_specs=[
            pl.BlockSpec((128, FD), lambda i: (i, 0)),
            pl.BlockSpec((1, FD), lambda i: (0, 0)),
        ],
        out_shape=[
            jax.ShapeDtypeStruct((NP, FD), jnp.float32),
            jax.ShapeDtypeStruct((1, FD), jnp.float32),
        ],
    )


def _make_head(ks):
    def body(pmax_ref, psum_ref, w1_ref, b1_ref, w2_ref, b2_ref,
             w3_ref, b3_ref, out_ref):
        h = None
        for l in range(5):
            mx = jnp.maximum(pmax_ref[l, 0, :], pmax_ref[l, 1, :])
            mean = (psum_ref[l, 0, :] + psum_ref[l, 1, :]) * (1.0 / ks[l])
            hl = jnp.concatenate([mx, mean])[None, :]
            h = hl if h is None else h + hl
        h = jnp.maximum(
            jnp.dot(h, w1_ref[...], preferred_element_type=jnp.float32)
            + b1_ref[...], 0.0)
        h = jnp.maximum(
            jnp.dot(h, w2_ref[...], preferred_element_type=jnp.float32)
            + b2_ref[...], 0.0)
        h = jnp.dot(h, w3_ref[...],
                    preferred_element_type=jnp.float32) + b3_ref[...]
        m = jnp.max(h, axis=-1, keepdims=True)
        sh = h - m
        out_ref[...] = sh - jnp.log(jnp.sum(jnp.exp(sh), axis=-1,
                                            keepdims=True))

    return pl.pallas_call(
        body,
        out_shape=jax.ShapeDtypeStruct((1, 2), jnp.float32),
    )


def kernel(x, edge_index, batch,
           conv1_W, conv1_b, pool1_w, conv2_W, conv2_b, pool2_w,
           conv3_W, conv3_b, pool3_w, conv4_W, conv4_b, pool4_w,
           conv5_W, conv5_b, pool5_w,
           lin1_W, lin1_b, lin2_W, lin2_b, lin3_W, lin3_b):
    convs = [(conv1_W, conv1_b, pool1_w), (conv2_W, conv2_b, pool2_w),
             (conv3_W, conv3_b, pool3_w), (conv4_W, conv4_b, pool4_w),
             (conv5_W, conv5_b, pool5_w)]
    rows = edge_index[0]
    cols = edge_index[1]
    pmaxes, psums = [], []
    xcur = jnp.pad(x, ((0, NPS[0] - NNODE), (0, 0)))
    for li, ((n, k), NP, NPn) in enumerate(zip(LAYER_SIZES, NPS, NPNS)):
        W, b, wp = convs[li]
        XL, winv = _make_tmm(n, NP)(xcur, W.T, b[None, :], wp[None, :])
        y, z, _xls = _make_s1(n, NP)(rows, cols, XL, wp)
        xcur, rows, cols, pmax, psum = _make_s2(n, k, NP, NPn)(
            z, y, rows, cols, winv)
        pmaxes.append(pmax)
        psums.append(psum)
    ks = [float(k) for (_, k) in LAYER_SIZES]
    out = _make_head(ks)(
        jnp.stack(pmaxes), jnp.stack(psums),
        lin1_W.T, lin1_b[None, :], lin2_W.T, lin2_b[None, :],
        lin3_W.T, lin3_b[None, :])
    return out


# speedup vs baseline: 1.0259x; 1.0259x over previous
import functools
import math

import jax
import jax.numpy as jnp
from jax import lax
from jax.experimental import pallas as pl
from jax.experimental.pallas import tpu as pltpu
from jax.experimental.pallas import tpu_sc as plsc

NC = 2
NS = 16
LN = 16
NNODE = 10000
EDG = 320000
FD = 128
POOL_RATIO = 0.8

_NEG = -3.0e38


def _pad1024(v):
    return ((v + 1023) // 1024) * 1024


def _layer_sizes():
    out = []
    n = NNODE
    for _ in range(5):
        k = int(math.ceil(POOL_RATIO * n))
        out.append((n, k))
        n = k
    return out


LAYER_SIZES = _layer_sizes()
NPS = [_pad1024(n + 1) for (n, _) in LAYER_SIZES]
NPNS = [_pad1024(k + 1) for (_, k) in LAYER_SIZES]

_GATHER_DNUMS = lax.GatherDimensionNumbers(
    offset_dims=(), collapsed_slice_dims=(0,), start_index_map=(0,))


def _take16(v, idx):
    return lax.gather(v, idx[:, None], _GATHER_DNUMS, (1,),
                      mode=lax.GatherScatterMode.PROMISE_IN_BOUNDS)


def _lane_splat(vec, j):
    return _take16(vec, jnp.full((LN,), j, jnp.int32))


def _hsum16v(v):
    io = lax.iota(jnp.int32, LN)
    for sh in (8, 4, 2, 1):
        v = v + _take16(v, (io + sh) & (LN - 1))
    return v


def _last16(v):
    return _lane_splat(v, LN - 1)


def _cumsum16(v):
    io = lax.iota(jnp.int32, LN)
    zero = jnp.zeros_like(v)
    for sh in (1, 2, 4, 8):
        shifted = jnp.where(io >= sh, _take16(v, jnp.maximum(io - sh, 0)),
                            zero)
        v = v + shifted
    return v


def _rsqrt_newton(d):
    bits = lax.bitcast_convert_type(d, jnp.int32)
    y = lax.bitcast_convert_type(
        jnp.int32(0x5F3759DF) - lax.shift_right_arithmetic(bits, 1),
        jnp.float32)
    half = 0.5 * d
    for _ in range(3):
        y = y * (1.5 - half * y * y)
    return y


def _mono_key(z):
    b = lax.bitcast_convert_type(z, jnp.uint32)
    sign = lax.shift_right_logical(b, jnp.uint32(31))
    xorv = jnp.uint32(0x80000000) + sign * jnp.uint32(0x7FFFFFFF)
    return b ^ xorv


def _make_s1(n, NP):
    cpt = NP // NS
    RC = 32
    EC = 160
    ECD = 400
    EPT = EDG // NS
    NCH = EPT // EC
    NCHD = EPT // ECD
    NQ = FD // LN
    mesh = plsc.VectorSubcoreMesh(core_axis_name="c", subcore_axis_name="s",
                                  num_cores=1, num_subcores=NS)

    @functools.partial(
        pl.kernel,
        out_type=(
            jax.ShapeDtypeStruct((NP, FD), jnp.float32),
            jax.ShapeDtypeStruct((NP,), jnp.float32),
            jax.ShapeDtypeStruct((NP, FD), jnp.float32),
        ),
        mesh=mesh,
        compiler_params=pltpu.CompilerParams(needs_layout_passes=False),
        scratch_types=[
            pltpu.VMEM_SHARED((NP, FD), jnp.float32),
            pltpu.VMEM_SHARED((NP,), jnp.float32),
            pltpu.VMEM((RC, FD), jnp.float32),
            pltpu.VMEM((cpt,), jnp.float32),
            pltpu.VMEM((RC,), jnp.float32),
            pltpu.VMEM((EC,), jnp.int32),
            pltpu.VMEM((EC,), jnp.int32),
            pltpu.VMEM((EC,), jnp.int32),
            pltpu.VMEM((EC,), jnp.int32),
            pltpu.VMEM((EC, FD), jnp.float32),
            pltpu.VMEM((EC, FD), jnp.float32),
            pltpu.VMEM((ECD,), jnp.int32),
            pltpu.VMEM((ECD,), jnp.float32),
            pltpu.VMEM((FD,), jnp.float32),
            pltpu.SemaphoreType.DMA,
            pltpu.SemaphoreType.DMA,
        ],
    )
    def s1(rows_hbm, cols_hbm, xl_hbm, w_hbm, y_hbm, z_hbm, xls_hbm,
           acc_sp, deg_sp, tile0_v, dis_v, zc0_v,
           ri0_v, ri1_v, ci0_v, ci1_v, g0_v, g1_v, rid_v, one_v, w_v,
           gsem, ssem):
        s = lax.axis_index("s")
        rbase = s * cpt
        ebase = s * EPT
        zeros16 = jnp.zeros((LN,), jnp.float32)
        ones16 = jnp.ones((LN,), jnp.float32)
        iota16 = lax.iota(jnp.int32, LN)

        def _fill0(i, _):
            zc0_v[pl.ds(i * LN, LN)] = zeros16
            return 0
        lax.fori_loop(0, RC // LN, _fill0, 0)

        def _zdeg(h, _):
            pltpu.sync_copy(zc0_v, deg_sp.at[pl.ds(rbase + h * RC, RC)])
            return 0
        lax.fori_loop(0, cpt // RC, _zdeg, 0)

        def _fill1(i, _):
            one_v[pl.ds(i * LN, LN)] = ones16
            return 0
        lax.fori_loop(0, ECD // LN, _fill1, 0)
        pltpu.sync_copy(w_hbm, w_v)
        plsc.subcore_barrier()

        def _deg_chunk(j, _):
            pltpu.sync_copy(rows_hbm.at[pl.ds(ebase + j * ECD, ECD)], rid_v)

            def _rw(i, _2):
                r = rid_v[pl.ds(i * LN, LN)]
                spread = n + 1 + ((i * LN + iota16) & 127)
                rid_v[pl.ds(i * LN, LN)] = jnp.where(r == n, spread, r)
                return 0
            lax.fori_loop(0, ECD // LN, _rw, 0)
            pltpu.sync_copy(one_v, deg_sp.at[rid_v], add=True)
            return 0
        lax.fori_loop(0, NCHD, _deg_chunk, 0)
        plsc.subcore_barrier()

        pltpu.sync_copy(deg_sp.at[pl.ds(rbase, cpt)], dis_v)

        def _dis(i, _):
            d = dis_v[pl.ds(i * LN, LN)]
            gid = rbase + i * LN + iota16
            d = d + jnp.where(gid < n, 1.0, 0.0)
            dis_v[pl.ds(i * LN, LN)] = _rsqrt_newton(d)
            return 0
        lax.fori_loop(0, cpt // LN, _dis, 0)

        def _stage(h, _):
            lb2 = h * RC
            rb2 = rbase + lb2
            pltpu.sync_copy(xl_hbm.at[pl.ds(rb2, RC)], tile0_v)

            def _scale(i2, _2):
                dvec = dis_v[pl.ds(lb2 + i2 * LN, LN)]

                def _srow(jj, _3):
                    i = i2 * LN + jj
                    sd = _lane_splat(dvec, jj)
                    for q in range(NQ):
                        v = tile0_v[i, pl.ds(q * LN, LN)]
                        tile0_v[i, pl.ds(q * LN, LN)] = v * sd
                    return 0
                lax.fori_loop(0, LN, _srow, 0)
                return 0
            lax.fori_loop(0, RC // LN, _scale, 0)
            pltpu.sync_copy(tile0_v, xls_hbm.at[pl.ds(rb2, RC)])
            pltpu.sync_copy(tile0_v, acc_sp.at[pl.ds(rb2, RC)])
            return 0
        lax.fori_loop(0, cpt // RC, _stage, 0)
        plsc.subcore_barrier()

        bufs = ((ri0_v, ci0_v, g0_v), (ri1_v, ci1_v, g1_v))

        def _load_rewrite(j, ri_b, ci_b):
            pltpu.sync_copy(rows_hbm.at[pl.ds(ebase + j * EC, EC)], ri_b)
            pltpu.sync_copy(cols_hbm.at[pl.ds(ebase + j * EC, EC)], ci_b)

            def _rw(i, _2):
                r = ri_b[pl.ds(i * LN, LN)]
                cc = ci_b[pl.ds(i * LN, LN)]
                spread = n + 1 + ((i * LN + iota16) & 127)
                m = r == n
                ri_b[pl.ds(i * LN, LN)] = jnp.where(m, spread, r)
                ci_b[pl.ds(i * LN, LN)] = jnp.where(m, spread, cc)
                return 0
            lax.fori_loop(0, EC // LN, _rw, 0)

        def _iter(j, cur, nxt):
            ri_b, ci_b, g_b = cur
            ri_n, ci_n, g_n = nxt
            pltpu.make_async_copy(xls_hbm.at[ri_b], g_b, gsem).wait()
            pltpu.async_copy(g_b, acc_sp.at[ci_b], ssem, add=True)

            @pl.when(j + 1 < NCH)
            def _pref():
                @pl.when(j >= 1)
                def _ws():
                    pltpu.make_async_copy(g_n, acc_sp.at[ci_n], ssem).wait()
                _load_rewrite(j + 1, ri_n, ci_n)
                pltpu.async_copy(xls_hbm.at[ri_n], g_n, gsem)

        _load_rewrite(0, ri0_v, ci0_v)
        pltpu.async_copy(xls_hbm.at[ri0_v], g0_v, gsem)

        def _agg_chunk(j, _):
            @pl.when(j % 2 == 0)
            def _even():
                _iter(j, bufs[0], bufs[1])

            @pl.when(j % 2 == 1)
            def _odd():
                _iter(j, bufs[1], bufs[0])
            return 0
        lax.fori_loop(0, NCH, _agg_chunk, 0)
        pltpu.make_async_copy(g0_v, acc_sp.at[ci0_v], ssem).wait()
        pltpu.make_async_copy(g1_v, acc_sp.at[ci1_v], ssem).wait()
        plsc.subcore_barrier()

        w_regs = [w_v[pl.ds(q * LN, LN)] for q in range(NQ)]

        def _final(h, _):
            lb2 = h * RC
            rb2 = rbase + lb2
            pltpu.sync_copy(acc_sp.at[pl.ds(rb2, RC)], tile0_v)

            def _fin(i2, _2):
                dvec = dis_v[pl.ds(lb2 + i2 * LN, LN)]

                def _frow(jj, zacc):
                    i = i2 * LN + jj
                    gid = rb2 + i
                    sd = _lane_splat(dvec, jj)
                    dot = zeros16
                    for q in range(NQ):
                        v = tile0_v[i, pl.ds(q * LN, LN)]
                        v = jnp.maximum(v * sd, 0.0)
                        tile0_v[i, pl.ds(q * LN, LN)] = v
                        dot = dot + v * w_regs[q]
                    zval = jnp.where(gid < n, _hsum16v(dot),
                                     jnp.full((LN,), _NEG, jnp.float32))
                    return jnp.where(iota16 == jj, zval, zacc)
                zacc = lax.fori_loop(0, LN, _frow,
                                     jnp.full((LN,), _NEG, jnp.float32))
                zc0_v[pl.ds(i2 * LN, LN)] = zacc
                return 0
            lax.fori_loop(0, RC // LN, _fin, 0)
            pltpu.sync_copy(tile0_v, y_hbm.at[pl.ds(rb2, RC)])
            pltpu.sync_copy(zc0_v, z_hbm.at[pl.ds(rb2, RC)])
            return 0
        lax.fori_loop(0, cpt // RC, _final, 0)

    return s1


def _make_s2(n, k, NP, NPn):
    cpk = NPn // (NC * NS)
    CC = 32
    NCC = cpk // CC
    CE = 2000
    EPT2 = EDG // (NC * NS)
    NCH2 = EPT2 // CE
    NQ = FD // LN
    mesh = plsc.VectorSubcoreMesh(core_axis_name="c", subcore_axis_name="s",
                                  num_cores=NC, num_subcores=NS)

    @functools.partial(
        pl.kernel,
        out_type=(
            jax.ShapeDtypeStruct((NPn, FD), jnp.float32),
            jax.ShapeDtypeStruct((EDG,), jnp.int32),
            jax.ShapeDtypeStruct((EDG,), jnp.int32),
            jax.ShapeDtypeStruct((NC, FD), jnp.float32),
            jax.ShapeDtypeStruct((NC, FD), jnp.float32),
        ),
        mesh=mesh,
        compiler_params=pltpu.CompilerParams(needs_layout_passes=False),
        scratch_types=[
            pltpu.VMEM_SHARED((NS, FD), jnp.float32),
            pltpu.VMEM_SHARED((NS, FD), jnp.float32),
            pltpu.VMEM((NP,), jnp.float32),
            pltpu.VMEM((NP,), jnp.uint32),
            pltpu.VMEM((4096,), jnp.int32),
            pltpu.VMEM((NPn,), jnp.int32),
            pltpu.VMEM((NP,), jnp.int32),
            pltpu.VMEM((cpk,), jnp.float32),
            pltpu.VMEM((CC, FD), jnp.float32),
            pltpu.VMEM((CE,), jnp.int32),
            pltpu.VMEM((CE,), jnp.int32),
            pltpu.VMEM((FD,), jnp.float32),
            pltpu.VMEM((NS, FD), jnp.float32),
            pltpu.VMEM((LN,), jnp.float32),
        ],
    )
    def s2(z_hbm, y_hbm, rows_hbm, cols_hbm, winv_hbm,
           xn_hbm, rn_hbm, cn_hbm, pmax_hbm, psum_hbm,
           mx_sp, sm_sp, z_v, key_v, subh_v, sel_v, map_v, sc_v,
           xb_v, er_v, ec_v, ro_v, mst_v, wv_v):
        c = lax.axis_index("c")
        s = lax.axis_index("s")
        w = s * NC + c
        iota16 = lax.iota(jnp.int32, LN)
        ones16i = jnp.ones((LN,), jnp.int32)
        zeros16i = jnp.zeros((LN,), jnp.int32)

        pltpu.sync_copy(z_hbm, z_v)
        pltpu.sync_copy(winv_hbm.at[0].at[pl.ds(0, LN)], wv_v)

        def _keys(i, _):
            key_v[pl.ds(i * LN, LN)] = _mono_key(z_v[pl.ds(i * LN, LN)])
            return 0
        lax.fori_loop(0, NP // LN, _keys, 0)

        er_v[pl.ds(0, LN)] = ones16i
        ones_m = er_v[pl.ds(0, LN)]

        prefix = jnp.zeros((LN,), jnp.uint32)
        below = jnp.zeros((LN,), jnp.int32)
        q_eq = jnp.zeros((LN,), jnp.int32)
        target = jnp.full((LN,), NP - k, jnp.int32)
        for p in range(4):
            shift = 24 - 8 * p

            def _zeroh(i, _):
                subh_v[pl.ds(i * LN, LN)] = zeros16i
                return 0
            lax.fori_loop(0, 4096 // LN, _zeroh, 0)

            pref = prefix

            def _hist(i, _, _p=p, _sh=shift, _pref=pref):
                kv = key_v[pl.ds(i * LN, LN)]
                field = lax.convert_element_type(
                    lax.shift_right_logical(kv, jnp.uint32(_sh))
                    & jnp.uint32(255), jnp.int32)
                addr = field * LN + iota16
                if _p == 0:
                    plsc.addupdate_scatter(subh_v, [addr], ones_m)
                else:
                    match = lax.shift_right_logical(
                        kv, jnp.uint32(_sh + 8)) == lax.shift_right_logical(
                            _pref, jnp.uint32(_sh + 8))
                    plsc.addupdate_scatter(subh_v, [addr], ones_m, mask=match)
                return 0
            lax.fori_loop(0, NP // LN, _hist, 0)

            def _walk(b, st):
                below_, vstar_, done_, q_ = st
                hv = subh_v[pl.ds(b * LN, LN)]
                sb = _hsum16v(hv)
                cross = jnp.logical_and(jnp.logical_not(done_),
                                        below_ + sb > target)
                vstar_ = jnp.where(cross, b, vstar_)
                q_ = jnp.where(cross, sb, q_)
                below_ = jnp.where(jnp.logical_or(done_, cross), below_,
                                   below_ + sb)
                done_ = jnp.logical_or(done_, cross)
                return below_, vstar_, done_, q_
            below, vstar, _, q_eq = lax.fori_loop(
                0, 256, _walk,
                (below, jnp.zeros((LN,), jnp.int32),
                 jnp.zeros((LN,), jnp.bool_), q_eq))
            prefix = prefix | lax.shift_left(
                lax.convert_element_type(vstar, jnp.uint32), jnp.uint32(shift))
        tkey = prefix
        n_gt = jnp.full((LN,), NP, jnp.int32) - below - q_eq
        need_eq = jnp.full((LN,), k, jnp.int32) - n_gt

        def _pref0(i, _):
            sel_v[pl.ds(i * LN, LN)] = zeros16i
            return 0
        lax.fori_loop(0, NPn // LN, _pref0, 0)

        def _compact(i, st):
            nsel, neq = st
            kv = key_v[pl.ds(i * LN, LN)]
            m_gt = kv > tkey
            m_eq = kv == tkey
            ceq = _cumsum16(jnp.where(m_eq, 1, 0))
            take = jnp.logical_or(
                m_gt, jnp.logical_and(m_eq, neq + ceq <= need_eq))
            ctk = _cumsum16(jnp.where(take, 1, 0))
            pos = nsel + ctk - 1
            plsc.store_scatter(sel_v, [pos], i * LN + iota16, mask=take)
            return nsel + _last16(ctk), neq + _last16(ceq)
        lax.fori_loop(0, NP // LN, _compact,
                      (jnp.zeros((LN,), jnp.int32),
                       jnp.zeros((LN,), jnp.int32)))

        winv = _lane_splat(wv_v[...], 0)
        rb = w * cpk

        def _score(i, _):
            pos = rb + i * LN + iota16
            posc = jnp.minimum(pos, NPn - 1)
            sidx = plsc.load_gather(sel_v, [posc])
            zv = plsc.load_gather(z_v, [sidx])
            u = zv * winv
            t = jnp.exp(-2.0 * jnp.abs(u))
            th = (1.0 - t) / (1.0 + t)
            sc = jnp.where(u < 0.0, -th, th)
            sc = jnp.where(pos < k, sc, 0.0)
            sc_v[pl.ds(i * LN, LN)] = sc
            return 0
        lax.fori_loop(0, cpk // LN, _score, 0)

        carry0 = tuple([jnp.full((LN,), _NEG, jnp.float32)] * NQ
                       + [jnp.zeros((LN,), jnp.float32)] * NQ)

        def _chunk(m, carry):
            base = rb + m * CC
            pltpu.sync_copy(y_hbm.at[sel_v.at[pl.ds(base, CC)]], xb_v)

            def _row(j, st_):
                st_ = list(st_)
                scvec = sc_v[pl.ds((m * CC + j) // LN * LN, LN)]
                sc = _lane_splat(scvec, j % LN)
                real = base + j < k
                for q in range(NQ):
                    v = xb_v[j, pl.ds(q * LN, LN)] * sc
                    xb_v[j, pl.ds(q * LN, LN)] = v
                    st_[q] = jnp.where(real, jnp.maximum(st_[q], v), st_[q])
                    st_[NQ + q] = jnp.where(real, st_[NQ + q] + v,
                                            st_[NQ + q])
                return tuple(st_)
            st = lax.fori_loop(0, CC, _row, carry)
            pltpu.sync_copy(xb_v, xn_hbm.at[pl.ds(base, CC)])
            return st
        carry = lax.fori_loop(0, NCC, _chunk, carry0)
        carry = list(carry)

        for q in range(NQ):
            ro_v[pl.ds(q * LN, LN)] = carry[q]
        pltpu.sync_copy(ro_v, mx_sp.at[s])
        for q in range(NQ):
            ro_v[pl.ds(q * LN, LN)] = carry[NQ + q]
        pltpu.sync_copy(ro_v, sm_sp.at[s])

        neg16 = jnp.full((LN,), -1, jnp.int32)

        def _minit(i, _):
            map_v[pl.ds(i * LN, LN)] = neg16
            return 0
        lax.fori_loop(0, NP // LN, _minit, 0)

        def _mset(i, _):
            sidx = sel_v[pl.ds(i * LN, LN)]
            newid = i * LN + iota16
            plsc.store_scatter(map_v, [sidx], newid, mask=newid < k)
            return 0
        lax.fori_loop(0, (k + LN - 1) // LN, _mset, 0)

        def _remap(j, _):
            eb = w * EPT2 + j * CE
            pltpu.sync_copy(rows_hbm.at[pl.ds(eb, CE)], er_v)
            pltpu.sync_copy(cols_hbm.at[pl.ds(eb, CE)], ec_v)

            def _rm(i, _2):
                r = er_v[pl.ds(i * LN, LN)]
                cc2 = ec_v[pl.ds(i * LN, LN)]
                mr = plsc.load_gather(map_v, [r])
                mc = plsc.load_gather(map_v, [cc2])
                valid = jnp.logical_and(mr >= 0, mc >= 0)
                er_v[pl.ds(i * LN, LN)] = jnp.where(valid, mr, k)
                ec_v[pl.ds(i * LN, LN)] = jnp.where(valid, mc, k)
                return 0
            lax.fori_loop(0, CE // LN, _rm, 0)
            pltpu.sync_copy(er_v, rn_hbm.at[pl.ds(eb, CE)])
            pltpu.sync_copy(ec_v, cn_hbm.at[pl.ds(eb, CE)])
            return 0
        lax.fori_loop(0, NCH2, _remap, 0)

        plsc.subcore_barrier()

        @pl.when(s == 0)
        def _merge():
            pltpu.sync_copy(mx_sp, mst_v)

            def _redmax(r, st_):
                st_ = list(st_)
                for q in range(NQ):
                    st_[q] = jnp.maximum(st_[q], mst_v[r, pl.ds(q * LN, LN)])
                return tuple(st_)
            accm = lax.fori_loop(
                0, NS, _redmax,
                tuple([jnp.full((LN,), _NEG, jnp.float32)] * NQ))
            for q in range(NQ):
                ro_v[pl.ds(q * LN, LN)] = accm[q]
            pltpu.sync_copy(ro_v, pmax_hbm.at[c])

            pltpu.sync_copy(sm_sp, mst_v)

            def _redsum(r, st_):
                st_ = list(st_)
                for q in range(NQ):
                    st_[q] = st_[q] + mst_v[r, pl.ds(q * LN, LN)]
                return tuple(st_)
            accs = lax.fori_loop(
                0, NS, _redsum,
                tuple([jnp.zeros((LN,), jnp.float32)] * NQ))
            for q in range(NQ):
                ro_v[pl.ds(q * LN, LN)] = accs[q]
            pltpu.sync_copy(ro_v, psum_hbm.at[c])

    return s2


def _make_tmm(n, NP):
    def body(x_ref, wt_ref, b_ref, wp_ref, out_ref, winv_ref):
        i = pl.program_id(0)
        acc = jnp.dot(x_ref[...], wt_ref[...],
                      preferred_element_type=jnp.float32) + b_ref[...]
        rid = i * 128 + lax.broadcasted_iota(jnp.int32, (128, FD), 0)
        out_ref[...] = jnp.where(rid < n, acc, 0.0)

        @pl.when(i == 0)
        def _():
            wp = wp_ref[...]
            winv_ref[...] = jnp.full((1, FD),
                                     lax.rsqrt(jnp.sum(wp * wp)), jnp.float32)

    return pl.pallas_call(
        body,
        grid=(NP // 128,),
        in_specs=[
            pl.BlockSpec((128, FD), lambda i: (i, 0)),
            pl.BlockSpec((FD, FD), lambda i: (0, 0)),
            pl.BlockSpec((1, FD), lambda i: (0, 0)),
            pl.BlockSpec((1, FD), lambda i: (0, 0)),
        ],
        out_specs=[
            pl.BlockSpec((128, FD), lambda i: (i, 0)),
            pl.BlockSpec((1, FD), lambda i: (0, 0)),
        ],
        out_shape=[
            jax.ShapeDtypeStruct((NP, FD), jnp.float32),
            jax.ShapeDtypeStruct((1, FD), jnp.float32),
        ],
    )


def _make_head(ks):
    def body(pmax_ref, psum_ref, w1_ref, b1_ref, w2_ref, b2_ref,
             w3_ref, b3_ref, out_ref):
        h = None
        for l in range(5):
            mx = jnp.maximum(pmax_ref[l, 0, :], pmax_ref[l, 1, :])
            mean = (psum_ref[l, 0, :] + psum_ref[l, 1, :]) * (1.0 / ks[l])
            hl = jnp.concatenate([mx, mean])[None, :]
            h = hl if h is None else h + hl
        h = jnp.maximum(
            jnp.dot(h, w1_ref[...], preferred_element_type=jnp.float32)
            + b1_ref[...], 0.0)
        h = jnp.maximum(
            jnp.dot(h, w2_ref[...], preferred_element_type=jnp.float32)
            + b2_ref[...], 0.0)
        h = jnp.dot(h, w3_ref[...],
                    preferred_element_type=jnp.float32) + b3_ref[...]
        m = jnp.max(h, axis=-1, keepdims=True)
        sh = h - m
        out_ref[...] = sh - jnp.log(jnp.sum(jnp.exp(sh), axis=-1,
                                            keepdims=True))

    return pl.pallas_call(
        body,
        out_shape=jax.ShapeDtypeStruct((1, 2), jnp.float32),
    )


def kernel(x, edge_index, batch,
           conv1_W, conv1_b, pool1_w, conv2_W, conv2_b, pool2_w,
           conv3_W, conv3_b, pool3_w, conv4_W, conv4_b, pool4_w,
           conv5_W, conv5_b, pool5_w,
           lin1_W, lin1_b, lin2_W, lin2_b, lin3_W, lin3_b):
    convs = [(conv1_W, conv1_b, pool1_w), (conv2_W, conv2_b, pool2_w),
             (conv3_W, conv3_b, pool3_w), (conv4_W, conv4_b, pool4_w),
             (conv5_W, conv5_b, pool5_w)]
    rows = edge_index[0]
    cols = edge_index[1]
    pmaxes, psums = [], []
    xcur = jnp.pad(x, ((0, NPS[0] - NNODE), (0, 0)))
    for li, ((n, k), NP, NPn) in enumerate(zip(LAYER_SIZES, NPS, NPNS)):
        W, b, wp = convs[li]
        XL, winv = _make_tmm(n, NP)(xcur, W.T, b[None, :], wp[None, :])
        y, z, _xls = _make_s1(n, NP)(rows, cols, XL, wp)
        xcur, rows, cols, pmax, psum = _make_s2(n, k, NP, NPn)(
            z, y, rows, cols, winv)
        pmaxes.append(pmax)
        psums.append(psum)
    ks = [float(k) for (_, k) in LAYER_SIZES]
    out = _make_head(ks)(
        jnp.stack(pmaxes), jnp.stack(psums),
        lin1_W.T, lin1_b[None, :], lin2_W.T, lin2_b[None, :],
        lin3_W.T, lin3_b[None, :])
    return out
